# baseline (speedup 1.0000x reference)
import functools

import jax
import jax.numpy as jnp
from jax import lax
from jax.experimental import pallas as pl
from jax.experimental.pallas import tpu as pltpu
from jax.experimental.pallas import tpu_sc as plsc

_KP1 = 6
_NB = 4096


def _normalize_and_score(q_ref, bank_ref):
    q = q_ref[...]
    qf = q[:, :-1]
    qn = qf / (jnp.sqrt(jnp.sum(qf * qf, axis=1, keepdims=True)) + 1e-12)
    qn = jnp.concatenate([qn, q[:, -1:]], axis=1)
    b = bank_ref[...]
    bf = b[:, :-1]
    bn = bf / (jnp.sqrt(jnp.sum(bf * bf, axis=1, keepdims=True)) + 1e-12)
    bn = jnp.concatenate([bn, b[:, -1:]], axis=1)
    return lax.dot_general(
        qn, bn, (((1,), (1,)), ((), ())),
        preferred_element_type=jnp.float32,
    )


def _merge(runv_ref, runi_ref, bv, bi):
    B = bv.shape[0]
    cv = jnp.concatenate([runv_ref[...], bv], axis=1)
    ci = jnp.concatenate([runi_ref[...], bi], axis=1)
    loc2 = lax.broadcasted_iota(jnp.int32, (B, 2 * _KP1), 1)
    nv, ni = [], []
    for _ in range(_KP1):
        m = jnp.max(cv, axis=1, keepdims=True)
        a = jnp.argmax(cv, axis=1).reshape(B, 1)
        sel = loc2 == a
        nv.append(m)
        ni.append(jnp.sum(jnp.where(sel, ci, 0), axis=1, keepdims=True))
        cv = jnp.where(sel, -jnp.inf, cv)
    runv_ref[...] = jnp.concatenate(nv, axis=1)
    runi_ref[...] = jnp.concatenate(ni, axis=1)


def _topk_fast_body(nblocks, q_ref, bank_ref, idx_ref, flag_ref,
                    runv_ref, runi_ref):
    pid = pl.program_id(0)
    B = q_ref.shape[0]

    @pl.when(pid == 0)
    def _init():
        runv_ref[...] = jnp.full((B, _KP1), -jnp.inf, jnp.float32)
        runi_ref[...] = jnp.zeros((B, _KP1), jnp.int32)
        flag_ref[...] = jnp.zeros((8, 128), jnp.int32)

    s = _normalize_and_score(q_ref, bank_ref)
    base = pid * bank_ref.shape[0]

    t = s
    bv, bi = [], []
    m = None
    bad = jnp.zeros((B, 1), jnp.int32)
    for j in range(_KP1):
        if j:
            eq = t == m
            bad = bad | (jnp.sum(eq.astype(jnp.int32), axis=1,
                                 keepdims=True) > 1).astype(jnp.int32)
            t = jnp.where(eq, -jnp.inf, t)
        m = jnp.max(t, axis=1, keepdims=True)
        a = jnp.argmax(t, axis=1).reshape(B, 1)
        bv.append(m)
        bi.append(a + base)
    bad = bad | (jnp.sum((t == m).astype(jnp.int32), axis=1,
                         keepdims=True) > 1).astype(jnp.int32)
    bv = jnp.concatenate(bv, axis=1)
    bi = jnp.concatenate(bi, axis=1)
    tie = jnp.max(bad)
    flag_ref[...] = flag_ref[...] | tie

    _merge(runv_ref, runi_ref, bv, bi)

    @pl.when(pid == nblocks - 1)
    def _emit():
        idx_ref[...] = runi_ref[...]


def _topk_exact_body(nblocks, q_ref, bank_ref, idx_ref, runv_ref, runi_ref):
    pid = pl.program_id(0)
    B = q_ref.shape[0]
    Nb = bank_ref.shape[0]

    @pl.when(pid == 0)
    def _init():
        runv_ref[...] = jnp.full((B, _KP1), -jnp.inf, jnp.float32)
        runi_ref[...] = jnp.zeros((B, _KP1), jnp.int32)

    s = _normalize_and_score(q_ref, bank_ref)
    base = pid * Nb
    loc = lax.broadcasted_iota(jnp.int32, (B, Nb), 1)
    bv, bi = [], []
    for _ in range(_KP1):
        m = jnp.max(s, axis=1, keepdims=True)
        a = jnp.argmax(s, axis=1).reshape(B, 1)
        bv.append(m)
        bi.append(a + base)
        s = jnp.where(loc == a, -jnp.inf, s)
    bv = jnp.concatenate(bv, axis=1)
    bi = jnp.concatenate(bi, axis=1)

    _merge(runv_ref, runi_ref, bv, bi)

    @pl.when(pid == nblocks - 1)
    def _emit():
        idx_ref[...] = runi_ref[...]


def _ext_inputs(queries, fea_bank):
    B = queries.shape[0]
    N = fea_bank.shape[0]
    nblocks = -(-N // _NB)
    npad = nblocks * _NB - N
    bank = jnp.pad(fea_bank, ((0, 0), (0, 1)))
    if npad:
        bank = jnp.pad(bank, ((0, npad), (0, 0)), constant_values=-4.0)
    q = jnp.pad(queries, ((0, 0), (0, 1)), constant_values=1.0)
    return q, bank, nblocks


def _topk_call(queries, fea_bank, interpret=False):
    B, D = queries.shape
    q, bank, nblocks = _ext_inputs(queries, fea_bank)
    scratch = [
        pltpu.VMEM((B, _KP1), jnp.float32),
        pltpu.VMEM((B, _KP1), jnp.int32),
    ]
    params = pltpu.CompilerParams(dimension_semantics=("arbitrary",))
    in_specs = [
        pl.BlockSpec((B, D + 1), lambda i: (0, 0)),
        pl.BlockSpec((_NB, D + 1), lambda i: (i, 0)),
    ]
    idx6, flag = pl.pallas_call(
        functools.partial(_topk_fast_body, nblocks),
        grid=(nblocks,),
        in_specs=in_specs,
        out_specs=[
            pl.BlockSpec((B, _KP1), lambda i: (0, 0)),
            pl.BlockSpec((8, 128), lambda i: (0, 0)),
        ],
        out_shape=[
            jax.ShapeDtypeStruct((B, _KP1), jnp.int32),
            jax.ShapeDtypeStruct((8, 128), jnp.int32),
        ],
        scratch_shapes=scratch,
        compiler_params=params,
        interpret=interpret,
    )(q, bank)

    exact = pl.pallas_call(
        functools.partial(_topk_exact_body, nblocks),
        grid=(nblocks,),
        in_specs=in_specs,
        out_specs=pl.BlockSpec((B, _KP1), lambda i: (0, 0)),
        out_shape=jax.ShapeDtypeStruct((B, _KP1), jnp.int32),
        scratch_shapes=scratch,
        compiler_params=params,
        interpret=interpret,
    )
    return lax.cond(flag[0, 0] > 0, lambda: exact(q, bank), lambda: idx6)


def _gather_call(table, idx_flat):
    BK = idx_flat.shape[0]
    Dp = table.shape[1]
    info = plsc.get_sparse_core_info()
    nw = info.num_cores * info.num_subcores
    b_per_w = BK // nw

    @functools.partial(
        pl.kernel,
        mesh=plsc.VectorSubcoreMesh(core_axis_name="c", subcore_axis_name="s"),
        out_type=jax.ShapeDtypeStruct((BK, Dp), jnp.float32),
        scratch_types=[
            pltpu.VMEM((b_per_w,), jnp.int32),
            pltpu.VMEM((b_per_w, Dp), jnp.float32),
            pltpu.SemaphoreType.DMA,
        ],
    )
    def k(table_hbm, idx_hbm, out_hbm, idx_v, rows_v, sem):
        wid = lax.axis_index("s") * info.num_cores + lax.axis_index("c")
        base = wid * b_per_w
        pltpu.sync_copy(idx_hbm.at[pl.ds(base, b_per_w)], idx_v)
        pltpu.async_copy(table_hbm.at[idx_v], rows_v, sem).wait()
        pltpu.sync_copy(rows_v, out_hbm.at[pl.ds(base, b_per_w)])

    return k(table, idx_flat)


def kernel(queries, fea_bank, score_bank):
    B = queries.shape[0]
    C = score_bank.shape[1]
    idx6 = _topk_call(queries, fea_bank)
    idx_near = idx6[:, 1:]
    K = _KP1 - 1
    table = jnp.pad(score_bank, ((0, 0), (0, 128 - C)))
    rows = _gather_call(table, idx_near.reshape(-1))
    score_near = rows[:, :C].reshape(B, K, C)
    return score_near, idx_near


# speedup vs baseline: 1.6628x; 1.6628x over previous
import functools

import jax
import jax.numpy as jnp
from jax import lax
from jax.experimental import pallas as pl
from jax.experimental.pallas import tpu as pltpu
from jax.experimental.pallas import tpu_sc as plsc

_KP1 = 6
_NB = 4096


def _normalize_and_score(q_ref, bank_ref):
    q = q_ref[...]
    qf = q[:, :-1]
    qn = qf / (jnp.sqrt(jnp.sum(qf * qf, axis=1, keepdims=True)) + 1e-12)
    qn = jnp.concatenate([qn, q[:, -1:]], axis=1)
    b = bank_ref[...]
    bf = b[:, :-1]
    bn = bf / (jnp.sqrt(jnp.sum(bf * bf, axis=1, keepdims=True)) + 1e-12)
    bn = jnp.concatenate([bn, b[:, -1:]], axis=1)
    return lax.dot_general(
        qn, bn, (((1,), (1,)), ((), ())),
        preferred_element_type=jnp.float32,
    )


def _merge(runv_ref, runi_ref, bv, bi):
    B = bv.shape[0]
    cv = jnp.concatenate([runv_ref[...], bv], axis=1)
    ci = jnp.concatenate([runi_ref[...], bi], axis=1)
    loc2 = lax.broadcasted_iota(jnp.int32, (B, 2 * _KP1), 1)
    nv, ni = [], []
    for _ in range(_KP1):
        m = jnp.max(cv, axis=1, keepdims=True)
        a = jnp.argmax(cv, axis=1).reshape(B, 1)
        sel = loc2 == a
        nv.append(m)
        ni.append(jnp.sum(jnp.where(sel, ci, 0), axis=1, keepdims=True))
        cv = jnp.where(sel, -jnp.inf, cv)
    runv_ref[...] = jnp.concatenate(nv, axis=1)
    runi_ref[...] = jnp.concatenate(ni, axis=1)


def _topk_exact_body(nblocks, q_ref, bank_ref, idx_ref, runv_ref, runi_ref):
    pid = pl.program_id(0)
    B = q_ref.shape[0]
    Nb = bank_ref.shape[0]

    @pl.when(pid == 0)
    def _init():
        runv_ref[...] = jnp.full((B, _KP1), -jnp.inf, jnp.float32)
        runi_ref[...] = jnp.zeros((B, _KP1), jnp.int32)

    s = _normalize_and_score(q_ref, bank_ref)
    base = pid * Nb
    loc = lax.broadcasted_iota(jnp.int32, (B, Nb), 1)
    bv, bi = [], []
    for _ in range(_KP1):
        m = jnp.max(s, axis=1, keepdims=True)
        a = jnp.argmax(s, axis=1).reshape(B, 1)
        bv.append(m)
        bi.append(a + base)
        s = jnp.where(loc == a, -jnp.inf, s)
    bv = jnp.concatenate(bv, axis=1)
    bi = jnp.concatenate(bi, axis=1)

    _merge(runv_ref, runi_ref, bv, bi)

    @pl.when(pid == nblocks - 1)
    def _emit():
        idx_ref[...] = runi_ref[...]


def _ext_inputs(queries, fea_bank):
    B = queries.shape[0]
    N = fea_bank.shape[0]
    nblocks = -(-N // _NB)
    npad = nblocks * _NB - N
    bank = jnp.pad(fea_bank, ((0, 0), (0, 1)))
    if npad:
        bank = jnp.pad(bank, ((0, npad), (0, 0)), constant_values=-4.0)
    q = jnp.pad(queries, ((0, 0), (0, 1)), constant_values=1.0)
    return q, bank, nblocks


def _topk_call(queries, fea_bank, interpret=False):
    B, D = queries.shape
    q, bank, nblocks = _ext_inputs(queries, fea_bank)
    return pl.pallas_call(
        functools.partial(_topk_exact_body, nblocks),
        grid=(nblocks,),
        in_specs=[
            pl.BlockSpec((B, D + 1), lambda i: (0, 0)),
            pl.BlockSpec((_NB, D + 1), lambda i: (i, 0)),
        ],
        out_specs=pl.BlockSpec((B, _KP1), lambda i: (0, 0)),
        out_shape=jax.ShapeDtypeStruct((B, _KP1), jnp.int32),
        scratch_shapes=[
            pltpu.VMEM((B, _KP1), jnp.float32),
            pltpu.VMEM((B, _KP1), jnp.int32),
        ],
        compiler_params=pltpu.CompilerParams(
            dimension_semantics=("arbitrary",)),
        interpret=interpret,
    )(q, bank)


def _gather_call(table, idx_flat):
    BK = idx_flat.shape[0]
    Dp = table.shape[1]
    info = plsc.get_sparse_core_info()
    nw = info.num_cores * info.num_subcores
    b_per_w = BK // nw

    @functools.partial(
        pl.kernel,
        mesh=plsc.VectorSubcoreMesh(core_axis_name="c", subcore_axis_name="s"),
        out_type=jax.ShapeDtypeStruct((BK, Dp), jnp.float32),
        scratch_types=[
            pltpu.VMEM((b_per_w,), jnp.int32),
            pltpu.VMEM((b_per_w, Dp), jnp.float32),
            pltpu.SemaphoreType.DMA,
        ],
    )
    def k(table_hbm, idx_hbm, out_hbm, idx_v, rows_v, sem):
        wid = lax.axis_index("s") * info.num_cores + lax.axis_index("c")
        base = wid * b_per_w
        pltpu.sync_copy(idx_hbm.at[pl.ds(base, b_per_w)], idx_v)
        pltpu.async_copy(table_hbm.at[idx_v], rows_v, sem).wait()
        pltpu.sync_copy(rows_v, out_hbm.at[pl.ds(base, b_per_w)])

    return k(table, idx_flat)


def kernel(queries, fea_bank, score_bank):
    B = queries.shape[0]
    C = score_bank.shape[1]
    idx6 = _topk_call(queries, fea_bank)
    idx_near = idx6[:, 1:]
    K = _KP1 - 1
    table = jnp.pad(score_bank, ((0, 0), (0, 128 - C)))
    rows = _gather_call(table, idx_near.reshape(-1))
    score_near = rows[:, :C].reshape(B, K, C)
    return score_near, idx_near
